# baseline (speedup 1.0000x reference)
import functools

import jax
import jax.numpy as jnp
from jax import lax
from jax.experimental import pallas as pl
from jax.experimental.pallas import tpu as pltpu
from jax.experimental.pallas import tpu_sc as plsc

_NUM_ACTIONS = 8
_NULL_IDX = _NUM_ACTIONS
_HIDDEN = 128
_BATCH = 16384
_TAB_ROWS = 16

_NC = 2
_NS = 16
_NW = _NC * _NS
_BPW = _BATCH // _NW
_LANES = 16
_GROUPS = _BPW // _LANES
_GF = _LANES * _HIDDEN


def _mlp_body(emb_ref, w1_ref, b1_ref, w2_ref, b2_ref, ids_ref, drop_ref,
              out_ref, tbases_ref):
    x = emb_ref[...]
    h = jnp.dot(x, w1_ref[...], preferred_element_type=jnp.float32)
    h = h + b1_ref[...]
    h = h * jax.nn.sigmoid(h)
    o = jnp.dot(h, w2_ref[...], preferred_element_type=jnp.float32)
    out_ref[...] = o + b2_ref[...]
    ids = ids_ref[...]
    drop = drop_ref[...]
    tbases_ref[...] = jnp.where(drop != 0, _NULL_IDX, ids) * _HIDDEN


_mlp_table = pl.pallas_call(
    _mlp_body,
    out_shape=[
        jax.ShapeDtypeStruct((_TAB_ROWS, _HIDDEN), jnp.float32),
        jax.ShapeDtypeStruct((_BATCH // _HIDDEN, _HIDDEN), jnp.int32),
    ],
)


def _gather_body(tbases_hbm, table_hbm, out_hbm, tbs_v, table_v, out_v,
                 sem_i, sem_t, sem_o):
    wid = lax.axis_index("s") * _NC + lax.axis_index("c")
    base = wid * _BPW
    cp_i = pltpu.async_copy(tbases_hbm.at[pl.ds(base, _BPW)], tbs_v, sem_i)
    cp_t = pltpu.async_copy(table_hbm, table_v, sem_t)
    cp_i.wait()
    cp_t.wait()

    obase_hbm = base * _HIDDEN

    @plsc.parallel_loop(0, _GROUPS, unroll=2)
    def group(g):
        sl = pl.ds(pl.multiple_of(g * _LANES, _LANES), _LANES)
        tbases = tbs_v[sl]
        obase = pl.multiple_of(g * _GF, _GF)
        for lane in range(_LANES):
            tb = pl.multiple_of(tbases[lane], _HIDDEN)
            ob = pl.multiple_of(obase + lane * _HIDDEN, _HIDDEN)
            for k in range(0, _HIDDEN, _LANES):
                out_v[pl.ds(ob + k, _LANES)] = table_v[pl.ds(tb + k, _LANES)]
        pltpu.async_copy(
            out_v.at[pl.ds(obase, _GF)],
            out_hbm.at[pl.ds(obase_hbm + obase, _GF)],
            sem_o)

    pltpu.make_async_copy(
        out_v, out_hbm.at[pl.ds(obase_hbm, _BPW * _HIDDEN)], sem_o).wait()


_gather = functools.partial(
    pl.kernel,
    out_type=jax.ShapeDtypeStruct((_BATCH * _HIDDEN,), jnp.float32),
    mesh=plsc.VectorSubcoreMesh(core_axis_name="c", subcore_axis_name="s",
                                num_cores=_NC, num_subcores=_NS),
    compiler_params=pltpu.CompilerParams(needs_layout_passes=False),
    scratch_types=[
        pltpu.VMEM((_BPW,), jnp.int32),
        pltpu.VMEM((_TAB_ROWS * _HIDDEN,), jnp.float32),
        pltpu.VMEM((_BPW * _HIDDEN,), jnp.float32),
        pltpu.SemaphoreType.DMA,
        pltpu.SemaphoreType.DMA,
        pltpu.SemaphoreType.DMA,
    ],
)(_gather_body)


@jax.jit
def kernel(action_ids, force_drop_ids, emb_table, W1, b1, W2, b2):
    emb_pad = jnp.zeros((_TAB_ROWS, _HIDDEN), jnp.float32).at[:_NUM_ACTIONS + 1].set(emb_table)
    table, tbases = _mlp_table(
        emb_pad, W1, b1[None, :], W2, b2[None, :],
        action_ids.astype(jnp.int32).reshape(_BATCH // _HIDDEN, _HIDDEN),
        force_drop_ids.astype(jnp.int32).reshape(_BATCH // _HIDDEN, _HIDDEN))
    out = _gather(tbases.reshape(-1), table.reshape(-1))
    return out.reshape(_BATCH, 1, _HIDDEN)


# speedup vs baseline: 1.1355x; 1.1355x over previous
import functools

import jax
import jax.numpy as jnp
from jax import lax
from jax.experimental import pallas as pl
from jax.experimental.pallas import tpu as pltpu
from jax.experimental.pallas import tpu_sc as plsc

_NUM_ACTIONS = 8
_NULL_IDX = _NUM_ACTIONS
_HIDDEN = 128
_BATCH = 16384
_TAB_ROWS = 16

_NC = 2
_NS = 16
_NW = _NC * _NS
_BPW = _BATCH // _NW
_LANES = 16
_GROUPS = _BPW // _LANES
_GF = _LANES * _HIDDEN


def _mlp_body(emb_ref, w1_ref, b1_ref, w2_ref, b2_ref, ids_ref, drop_ref,
              out_ref, tbases_ref):
    x = jnp.concatenate(
        [emb_ref[...],
         jnp.zeros((_TAB_ROWS - _NUM_ACTIONS - 1, _HIDDEN), jnp.float32)], 0)
    h = jnp.dot(x, w1_ref[...], preferred_element_type=jnp.float32)
    h = h + b1_ref[...]
    h = h * jax.nn.sigmoid(h)
    o = jnp.dot(h, w2_ref[...], preferred_element_type=jnp.float32)
    out_ref[...] = o + b2_ref[...]
    ids = ids_ref[...]
    drop = drop_ref[...]
    tbases_ref[...] = jnp.where(drop != 0, _NULL_IDX, ids) * _HIDDEN


_mlp_table = pl.pallas_call(
    _mlp_body,
    out_shape=[
        jax.ShapeDtypeStruct((_TAB_ROWS, _HIDDEN), jnp.float32),
        jax.ShapeDtypeStruct((_BATCH // _HIDDEN, _HIDDEN), jnp.int32),
    ],
)


def _gather_body(tbases_hbm, table_hbm, out_hbm, tbs_v, table_v, out_v,
                 sem_i, sem_t, sem_o):
    wid = lax.axis_index("s") * _NC + lax.axis_index("c")
    base = wid * _BPW
    cp_i = pltpu.async_copy(tbases_hbm.at[pl.ds(base, _BPW)], tbs_v, sem_i)
    cp_t = pltpu.async_copy(table_hbm, table_v, sem_t)
    cp_i.wait()
    cp_t.wait()

    obase_hbm = base * _HIDDEN

    @plsc.parallel_loop(0, _GROUPS)
    def group(g):
        sl = pl.ds(pl.multiple_of(g * _LANES, _LANES), _LANES)
        tbases = tbs_v[sl]
        obase = pl.multiple_of(g * _GF, _GF)
        for lane in range(_LANES):
            tb = pl.multiple_of(tbases[lane], _HIDDEN)
            ob = pl.multiple_of(obase + lane * _HIDDEN, _HIDDEN)
            for k in range(0, _HIDDEN, _LANES):
                out_v[pl.ds(ob + k, _LANES)] = table_v[pl.ds(tb + k, _LANES)]
        pltpu.async_copy(
            out_v.at[pl.ds(obase, _GF)],
            out_hbm.at[pl.ds(obase_hbm + obase, _GF)],
            sem_o)

    pltpu.make_async_copy(
        out_v, out_hbm.at[pl.ds(obase_hbm, _BPW * _HIDDEN)], sem_o).wait()


_gather = functools.partial(
    pl.kernel,
    out_type=jax.ShapeDtypeStruct((_BATCH * _HIDDEN,), jnp.float32),
    mesh=plsc.VectorSubcoreMesh(core_axis_name="c", subcore_axis_name="s",
                                num_cores=_NC, num_subcores=_NS),
    compiler_params=pltpu.CompilerParams(needs_layout_passes=False),
    scratch_types=[
        pltpu.VMEM((_BPW,), jnp.int32),
        pltpu.VMEM((_TAB_ROWS * _HIDDEN,), jnp.float32),
        pltpu.VMEM((_BPW * _HIDDEN,), jnp.float32),
        pltpu.SemaphoreType.DMA,
        pltpu.SemaphoreType.DMA,
        pltpu.SemaphoreType.DMA,
    ],
)(_gather_body)


@jax.jit
def kernel(action_ids, force_drop_ids, emb_table, W1, b1, W2, b2):
    table, tbases = _mlp_table(
        emb_table, W1, b1[None, :], W2, b2[None, :],
        action_ids.astype(jnp.int32).reshape(_BATCH // _HIDDEN, _HIDDEN),
        force_drop_ids.astype(jnp.int32).reshape(_BATCH // _HIDDEN, _HIDDEN))
    out = _gather(tbases.reshape(-1), table.reshape(-1))
    return out.reshape(_BATCH, 1, _HIDDEN)
